# baseline (speedup 1.0000x reference)
import jax
import jax.numpy as jnp
from jax.experimental import pallas as pl
from jax.experimental.pallas import tpu as pltpu


def _vq_body(z_ref, cb_ref, cth_ref, zq_ref, idx_ref, prev_ref):
    i = pl.program_id(0)
    nsteps = pl.num_programs(0)
    dn = (((1,), (0,)), ((), ()))
    k_codes = cb_ref.shape[0]

    @pl.when(i > 0)
    def _gather_prev():
        idxp = prev_ref[...]
        iig = jax.lax.broadcasted_iota(jnp.int32, (k_codes, idxp.shape[1]), 0)
        oh = (iig == idxp).astype(jnp.bfloat16)
        zq = jax.lax.dot_general(cth_ref[...], oh, dn,
                                 preferred_element_type=jnp.float32)
        zq_ref[0] = zq.reshape(zq_ref.shape[1:])
        idx_ref[0] = idxp

    @pl.when(i < nsteps - 1)
    def _score_cur():
        zb = z_ref[0].reshape(z_ref.shape[1], -1)
        cb = cb_ref[...]
        s = jax.lax.dot_general(cb, zb, dn, preferred_element_type=jnp.float32)
        znorm = jnp.sum(zb * zb, axis=0, keepdims=True)
        cnorm = jnp.sum(cb * cb, axis=1, keepdims=True)
        d = (znorm - 2.0 * s) + cnorm

        minv = jnp.min(d, axis=0, keepdims=True)
        ii = jax.lax.broadcasted_iota(jnp.int32, d.shape, 0)
        idx = jnp.min(jnp.where(d == minv, ii, k_codes), axis=0, keepdims=True)
        prev_ref[...] = idx


def kernel(z, codebook):
    b, c, h, w = z.shape
    hw = h * w
    k = codebook.shape[0]
    ct_hi = codebook.T.astype(jnp.bfloat16)

    zq4, idx3 = pl.pallas_call(
        _vq_body,
        grid=(b + 1,),
        in_specs=[
            pl.BlockSpec((1, c, h, w), lambda i: (jnp.minimum(i, b - 1), 0, 0, 0)),
            pl.BlockSpec((k, c), lambda i: (0, 0)),
            pl.BlockSpec((c, k), lambda i: (0, 0)),
        ],
        out_specs=[
            pl.BlockSpec((1, c, h, w), lambda i: (jnp.maximum(i - 1, 0), 0, 0, 0)),
            pl.BlockSpec((1, 1, hw), lambda i: (jnp.maximum(i - 1, 0), 0, 0)),
        ],
        out_shape=[
            jax.ShapeDtypeStruct((b, c, h, w), jnp.float32),
            jax.ShapeDtypeStruct((b, 1, hw), jnp.int32),
        ],
        scratch_shapes=[pltpu.VMEM((1, hw), jnp.int32)],
    )(z, codebook, ct_hi)
    return zq4, idx3.reshape(b, hw)


# speedup vs baseline: 3.7505x; 3.7505x over previous
import jax
import jax.numpy as jnp
from jax.experimental import pallas as pl


def _vq_body(zt_ref, ct_ref, cbbf_ref, zq_ref, idx_ref):
    zf = zt_ref[0]
    ct = ct_ref[...]
    k_codes = ct.shape[1]
    dn = (((1,), (0,)), ((), ()))

    s = jax.lax.dot_general(zf, ct, dn, preferred_element_type=jnp.float32)
    znorm = jnp.sum(zf * zf, axis=1, keepdims=True)
    cnorm = jnp.sum(ct * ct, axis=0, keepdims=True)
    d = (znorm - 2.0 * s) + cnorm

    minv = jnp.min(d, axis=1, keepdims=True)
    ii = jax.lax.broadcasted_iota(jnp.int32, d.shape, 1)
    idx = jnp.min(jnp.where(d == minv, ii, k_codes), axis=1, keepdims=True)

    oh = (ii == idx).astype(jnp.bfloat16)
    zq = jax.lax.dot_general(oh, cbbf_ref[...], dn,
                             preferred_element_type=jnp.float32)
    zq_ref[0] = zq
    idx_ref[0] = idx.reshape(1, idx.shape[0])


def kernel(z, codebook):
    b, c, h, w = z.shape
    hw = h * w
    k = codebook.shape[0]
    zt = z.transpose(0, 2, 3, 1).reshape(b, hw, c)
    ct = codebook.T
    cb_bf = codebook.astype(jnp.bfloat16)

    zq3, idx3 = pl.pallas_call(
        _vq_body,
        grid=(b,),
        in_specs=[
            pl.BlockSpec((1, hw, c), lambda i: (i, 0, 0)),
            pl.BlockSpec((c, k), lambda i: (0, 0)),
            pl.BlockSpec((k, c), lambda i: (0, 0)),
        ],
        out_specs=[
            pl.BlockSpec((1, hw, c), lambda i: (i, 0, 0)),
            pl.BlockSpec((1, 1, hw), lambda i: (i, 0, 0)),
        ],
        out_shape=[
            jax.ShapeDtypeStruct((b, hw, c), jnp.float32),
            jax.ShapeDtypeStruct((b, 1, hw), jnp.int32),
        ],
    )(zt, ct, cb_bf)
    zq = zq3.reshape(b, h, w, c).transpose(0, 3, 1, 2)
    return zq, idx3.reshape(b, hw)
